# baseline (speedup 1.0000x reference)
import jax
import jax.numpy as jnp
from jax import lax
from jax.experimental import pallas as pl
from jax.experimental.pallas import tpu as pltpu
from jax.experimental.pallas import tpu_sc as plsc

N = 10000
N_PAD = 10240
E = 320000
D_IN = 128
D_HID = 128
D_OUT = 40
D_OUT_PAD = 48

NUM_CORES = 2
NUM_SUBCORES = 16
NW = NUM_CORES * NUM_SUBCORES

KB = 80

EPT_D = E // NW
NB_D = EPT_D // KB
RPT = N_PAD // NUM_SUBCORES
ZCH = 128

N_HALF = N_PAD // 2
ACC_ROWS = N_HALF + 16
EPT_A = 20480
E_PAD = EPT_A * NUM_SUBCORES
NCH = 10
CH = EPT_A // NCH
KB2 = 128
CAP = 20992
OPT = N_HALF // NUM_SUBCORES
ZCH_A = 32

_MESH = plsc.VectorSubcoreMesh(core_axis_name="c", subcore_axis_name="s")


def _zero_fill(ref, rows, cols):
    lanes = cols // 16
    zeros16 = jnp.zeros((16,), jnp.float32)

    def body(i, carry):
        r = i // lanes
        c = (i % lanes) * 16
        ref[r, pl.ds(c, 16)] = zeros16
        return carry

    lax.fori_loop(0, rows * lanes, body, 0)


def _deg_kernel(dst3, out, dst_v, ones_v, zbuf, acc, sem):
    cid = lax.axis_index("c")
    sid = lax.axis_index("s")
    wid = cid * NUM_SUBCORES + sid

    _zero_fill(zbuf, ZCH, 16)
    ones16 = jnp.ones((16,), jnp.float32)

    def fill_ones(i, carry):
        ones_v[i, :] = ones16
        return carry

    lax.fori_loop(0, KB, fill_ones, 0)

    row0 = sid * RPT

    def zchunk(j, carry):
        pltpu.sync_copy(zbuf, acc.at[pl.ds(row0 + j * ZCH, ZCH)])
        return carry

    lax.fori_loop(0, RPT // ZCH, zchunk, 0)
    plsc.subcore_barrier()

    pltpu.sync_copy(dst3.at[wid], dst_v)

    def batch(j, carry):
        pltpu.sync_copy(ones_v, acc.at[dst_v.at[j]], add=True)
        return carry

    lax.fori_loop(0, NB_D, batch, 0)
    plsc.subcore_barrier()

    def ochunk(j, carry):
        r = row0 + j * ZCH
        pltpu.sync_copy(acc.at[pl.ds(r, ZCH)], out.at[cid, pl.ds(r, ZCH)])
        return carry

    lax.fori_loop(0, RPT // ZCH, ochunk, 0)


def _agg_kernel(mp, src3, dst3, out, sall, dall, raw_s, raw_d,
                rows0, zbuf, acc, sem0):
    cid = lax.axis_index("c")
    sid = lax.axis_index("s")
    lo = cid * N_HALF
    trash = N_HALF + sid

    _zero_fill(zbuf, ZCH_A, D_HID)

    def zchunk(j, carry):
        pltpu.sync_copy(zbuf, acc.at[pl.ds((sid * 10 + j) * ZCH_A, ZCH_A)])
        return carry

    lax.fori_loop(0, N_HALF // ZCH_A // NUM_SUBCORES, zchunk, 0)

    @pl.when(sid == 0)
    def _():
        pltpu.sync_copy(zbuf.at[pl.ds(0, 16)], acc.at[pl.ds(N_HALF, 16)])

    plsc.subcore_barrier()

    def chunk(ch, carry):
        pltpu.sync_copy(src3.at[sid, pl.ds(ch * (CH // 128), CH // 128)], raw_s)
        pltpu.sync_copy(dst3.at[sid, pl.ds(ch * (CH // 128), CH // 128)], raw_d)

        def step(i, carry):
            for sub in range(8):
                s16 = raw_s[i, pl.ds(sub * 16, 16)]
                d16 = raw_d[i, pl.ds(sub * 16, 16)]
                t = d16 - lo
                ok = (t >= 0) & (t < N_HALF)
                sall[ch * (CH // 128) + i, pl.ds(sub * 16, 16)] = s16
                dall[ch * (CH // 128) + i, pl.ds(sub * 16, 16)] = \
                    jnp.where(ok, t, trash)
            return carry

        return lax.fori_loop(0, CH // 128, step, carry)

    lax.fori_loop(0, NCH, chunk, 0)
    nb = EPT_A // KB2

    def pipe(j, carry):
        pltpu.async_copy(mp.at[sall.at[j]], rows0, sem0).wait()
        pltpu.sync_copy(rows0, acc.at[dall.at[j]], add=True)
        return carry

    lax.fori_loop(0, nb, pipe, 0)

    plsc.subcore_barrier()
    r = sid * OPT
    pltpu.sync_copy(acc.at[pl.ds(r, OPT)], out.at[pl.ds(lo + r, OPT)])


_deg = pl.kernel(
    _deg_kernel,
    out_type=jax.ShapeDtypeStruct((NUM_CORES, N_PAD, 16), jnp.float32),
    mesh=_MESH,
    scratch_types=[
        pltpu.VMEM((NB_D, KB), jnp.int32),
        pltpu.VMEM((KB, 16), jnp.float32),
        pltpu.VMEM((ZCH, 16), jnp.float32),
        pltpu.VMEM_SHARED((N_PAD, 16), jnp.float32),
        pltpu.SemaphoreType.DMA,
    ],
)

_agg = pl.kernel(
    _agg_kernel,
    out_type=jax.ShapeDtypeStruct((N_PAD, D_HID), jnp.float32),
    mesh=_MESH,
    scratch_types=[
        pltpu.VMEM((EPT_A // 128, 128), jnp.int32),
        pltpu.VMEM((EPT_A // 128, 128), jnp.int32),
        pltpu.VMEM((CH // 128, 128), jnp.int32),
        pltpu.VMEM((CH // 128, 128), jnp.int32),
        pltpu.VMEM((KB2, D_HID), jnp.float32),
        pltpu.VMEM((ZCH_A, D_HID), jnp.float32),
        pltpu.VMEM_SHARED((ACC_ROWS, D_HID), jnp.float32),
        pltpu.SemaphoreType.DMA,
    ],
)


BR = 1024


def _dis_from(degp_ref):
    deg = degp_ref[0, :, 0:1] + degp_ref[1, :, 0:1] + 1.0
    return lax.rsqrt(deg)


def _tc_a_body(x_ref, w_ref, degp_ref, m1p_ref):
    dis = _dis_from(degp_ref)
    m1p_ref[...] = dis * jnp.dot(x_ref[...], w_ref[...],
                                 preferred_element_type=jnp.float32)


def _tc_b_body(a1_ref, m1p_ref, degp_ref, b1_ref, hp_ref):
    dis = _dis_from(degp_ref)
    h = jnp.maximum(dis * (a1_ref[...] + m1p_ref[...]) + b1_ref[...], 0.0)
    hp_ref[...] = dis * h


def _tc_c_body(a2_ref, hp_ref, degp_ref, b2_ref, w2_ref, o_ref):
    dis = _dis_from(degp_ref)
    g = dis * (a2_ref[...] + hp_ref[...])
    o_ref[...] = jnp.dot(g, w2_ref[...], preferred_element_type=jnp.float32) \
        + b2_ref[...]


def _rows_spec(d):
    return pl.BlockSpec((BR, d), lambda i: (i, 0))


def _pair_spec(d):
    return pl.BlockSpec((2, BR, d), lambda i: (0, i, 0))


def _full_spec(shape):
    return pl.BlockSpec(shape, lambda i: tuple(0 for _ in shape))


_GRID = (N_PAD // BR,)

_tc_a = pl.pallas_call(
    _tc_a_body,
    grid=_GRID,
    in_specs=[_rows_spec(D_IN), _full_spec((D_IN, D_HID)), _pair_spec(16)],
    out_specs=_rows_spec(D_HID),
    out_shape=jax.ShapeDtypeStruct((N_PAD, D_HID), jnp.float32),
)

_tc_b = pl.pallas_call(
    _tc_b_body,
    grid=_GRID,
    in_specs=[_rows_spec(D_HID), _rows_spec(D_HID), _pair_spec(16),
              _full_spec((1, D_HID))],
    out_specs=_rows_spec(D_HID),
    out_shape=jax.ShapeDtypeStruct((N_PAD, D_HID), jnp.float32),
)

_tc_c = pl.pallas_call(
    _tc_c_body,
    grid=_GRID,
    in_specs=[_rows_spec(D_HID), _rows_spec(D_HID), _pair_spec(16),
              _full_spec((1, D_OUT_PAD)), _full_spec((D_HID, D_OUT_PAD))],
    out_specs=_rows_spec(D_OUT_PAD),
    out_shape=jax.ShapeDtypeStruct((N_PAD, D_OUT_PAD), jnp.float32),
)


def kernel(x, edge_index, W1, b1, W2, b2):
    xp = jnp.pad(x, ((0, N_PAD - N), (0, 0)))
    dst_d = edge_index[1].reshape(NW, NB_D, KB)
    pad_n = E_PAD - E
    src_a = jnp.concatenate(
        [edge_index[0], jnp.zeros((pad_n,), jnp.int32)]).reshape(
            NUM_SUBCORES, EPT_A // 128, 128)
    dst_a = jnp.concatenate(
        [edge_index[1], jnp.full((pad_n,), 1 << 20, jnp.int32)]).reshape(
            NUM_SUBCORES, EPT_A // 128, 128)
    w2p = jnp.pad(W2, ((0, 0), (0, D_OUT_PAD - D_OUT)))
    b1r = b1.reshape(1, D_HID)
    b2r = jnp.pad(b2, (0, D_OUT_PAD - D_OUT)).reshape(1, D_OUT_PAD)

    degp = _deg(dst_d)
    m1p = _tc_a(xp, W1, degp)
    a1 = _agg(m1p, src_a, dst_a)
    hp = _tc_b(a1, m1p, degp, b1r)
    a2 = _agg(hp, src_a, dst_a)
    o48 = _tc_c(a2, hp, degp, b2r, w2p)
    return o48[:N, :D_OUT]


# speedup vs baseline: 2.0015x; 2.0015x over previous
import jax
import jax.numpy as jnp
from jax import lax
from jax.experimental import pallas as pl
from jax.experimental.pallas import tpu as pltpu
from jax.experimental.pallas import tpu_sc as plsc

N = 10000
N_PAD = 10240
E = 320000
D_IN = 128
D_HID = 128
D_OUT = 40
D_OUT_PAD = 48

NUM_CORES = 2
NUM_SUBCORES = 16
NW = NUM_CORES * NUM_SUBCORES

KB = 80

EPT_D = E // NW
NB_D = EPT_D // KB
RPT = N_PAD // NUM_SUBCORES
ZCH = 128

N_HALF = N_PAD // 2
ACC_ROWS = N_HALF + 16
EPT_A = E // NUM_SUBCORES
NB_A = EPT_A // KB
OPT = N_HALF // NUM_SUBCORES
ZCH_A = 64

_MESH = plsc.VectorSubcoreMesh(core_axis_name="c", subcore_axis_name="s")


def _zero_fill(ref, rows, cols):
    lanes = cols // 16
    zeros16 = jnp.zeros((16,), jnp.float32)

    def body(i, carry):
        r = i // lanes
        c = (i % lanes) * 16
        ref[r, pl.ds(c, 16)] = zeros16
        return carry

    lax.fori_loop(0, rows * lanes, body, 0)


def _deg_kernel(dst3, out, dst_v, ones_v, zbuf, acc, sem):
    cid = lax.axis_index("c")
    sid = lax.axis_index("s")
    wid = cid * NUM_SUBCORES + sid

    _zero_fill(zbuf, ZCH, 16)
    ones16 = jnp.ones((16,), jnp.float32)

    def fill_ones(i, carry):
        ones_v[i, :] = ones16
        return carry

    lax.fori_loop(0, KB, fill_ones, 0)

    row0 = sid * RPT

    def zchunk(j, carry):
        pltpu.sync_copy(zbuf, acc.at[pl.ds(row0 + j * ZCH, ZCH)])
        return carry

    lax.fori_loop(0, RPT // ZCH, zchunk, 0)
    plsc.subcore_barrier()

    pltpu.sync_copy(dst3.at[wid], dst_v)

    def batch(j, carry):
        pltpu.sync_copy(ones_v, acc.at[dst_v.at[j]], add=True)
        return carry

    lax.fori_loop(0, NB_D, batch, 0)
    plsc.subcore_barrier()

    def ochunk(j, carry):
        r = row0 + j * ZCH
        pltpu.sync_copy(acc.at[pl.ds(r, ZCH)], out.at[cid, pl.ds(r, ZCH)])
        return carry

    lax.fori_loop(0, RPT // ZCH, ochunk, 0)


def _agg_kernel(mp, src3, dst3, out, src_v, dst_v, dstp, rows_v, zbuf, acc, sem):
    cid = lax.axis_index("c")
    sid = lax.axis_index("s")
    lo = cid * N_HALF
    trash = N_HALF + sid

    _zero_fill(zbuf, ZCH_A, D_HID)

    def zchunk(j, carry):
        pltpu.sync_copy(zbuf, acc.at[pl.ds((sid * 5 + j) * ZCH_A, ZCH_A)])
        return carry

    lax.fori_loop(0, N_HALF // ZCH_A // NUM_SUBCORES, zchunk, 0)

    @pl.when(sid == 0)
    def _():
        pltpu.sync_copy(zbuf.at[pl.ds(0, 16)], acc.at[pl.ds(N_HALF, 16)])

    plsc.subcore_barrier()

    pltpu.sync_copy(src3.at[sid], src_v)
    pltpu.sync_copy(dst3.at[sid], dst_v)

    def batch(j, carry):
        pltpu.async_copy(mp.at[src_v.at[j]], rows_v, sem).wait()
        for c in range(KB // 16):
            d = dst_v[j, pl.ds(c * 16, 16)]
            t = d - lo
            ok = (t >= 0) & (t < N_HALF)
            dstp[0, pl.ds(c * 16, 16)] = jnp.where(ok, t, trash)
        pltpu.sync_copy(rows_v, acc.at[dstp.at[0]], add=True)
        return carry

    lax.fori_loop(0, NB_A, batch, 0)
    plsc.subcore_barrier()

    r = sid * OPT
    pltpu.sync_copy(acc.at[pl.ds(r, OPT)], out.at[pl.ds(lo + r, OPT)])


_deg = pl.kernel(
    _deg_kernel,
    out_type=jax.ShapeDtypeStruct((NUM_CORES, N_PAD, 16), jnp.float32),
    mesh=_MESH,
    scratch_types=[
        pltpu.VMEM((NB_D, KB), jnp.int32),
        pltpu.VMEM((KB, 16), jnp.float32),
        pltpu.VMEM((ZCH, 16), jnp.float32),
        pltpu.VMEM_SHARED((N_PAD, 16), jnp.float32),
        pltpu.SemaphoreType.DMA,
    ],
)

_agg = pl.kernel(
    _agg_kernel,
    out_type=jax.ShapeDtypeStruct((N_PAD, D_HID), jnp.float32),
    mesh=_MESH,
    scratch_types=[
        pltpu.VMEM((NB_A, KB), jnp.int32),
        pltpu.VMEM((NB_A, KB), jnp.int32),
        pltpu.VMEM((8, KB), jnp.int32),
        pltpu.VMEM((KB, D_HID), jnp.float32),
        pltpu.VMEM((ZCH_A, D_HID), jnp.float32),
        pltpu.VMEM_SHARED((ACC_ROWS, D_HID), jnp.float32),
        pltpu.SemaphoreType.DMA,
    ],
)


BR = 1024


def _dis_from(degp_ref):
    deg = degp_ref[0, :, 0:1] + degp_ref[1, :, 0:1] + 1.0
    return lax.rsqrt(deg)


def _tc_a_body(x_ref, w_ref, degp_ref, m1p_ref):
    dis = _dis_from(degp_ref)
    m1p_ref[...] = dis * jnp.dot(x_ref[...], w_ref[...],
                                 preferred_element_type=jnp.float32)


def _tc_b_body(a1_ref, m1p_ref, degp_ref, b1_ref, hp_ref):
    dis = _dis_from(degp_ref)
    h = jnp.maximum(dis * (a1_ref[...] + m1p_ref[...]) + b1_ref[...], 0.0)
    hp_ref[...] = dis * h


def _tc_c_body(a2_ref, hp_ref, degp_ref, b2_ref, w2_ref, o_ref):
    dis = _dis_from(degp_ref)
    g = dis * (a2_ref[...] + hp_ref[...])
    o_ref[...] = jnp.dot(g, w2_ref[...], preferred_element_type=jnp.float32) \
        + b2_ref[...]


def _rows_spec(d):
    return pl.BlockSpec((BR, d), lambda i: (i, 0))


def _pair_spec(d):
    return pl.BlockSpec((2, BR, d), lambda i: (0, i, 0))


def _full_spec(shape):
    return pl.BlockSpec(shape, lambda i: tuple(0 for _ in shape))


_GRID = (N_PAD // BR,)

_tc_a = pl.pallas_call(
    _tc_a_body,
    grid=_GRID,
    in_specs=[_rows_spec(D_IN), _full_spec((D_IN, D_HID)), _pair_spec(16)],
    out_specs=_rows_spec(D_HID),
    out_shape=jax.ShapeDtypeStruct((N_PAD, D_HID), jnp.float32),
)

_tc_b = pl.pallas_call(
    _tc_b_body,
    grid=_GRID,
    in_specs=[_rows_spec(D_HID), _rows_spec(D_HID), _pair_spec(16),
              _full_spec((1, D_HID))],
    out_specs=_rows_spec(D_HID),
    out_shape=jax.ShapeDtypeStruct((N_PAD, D_HID), jnp.float32),
)

_tc_c = pl.pallas_call(
    _tc_c_body,
    grid=_GRID,
    in_specs=[_rows_spec(D_HID), _rows_spec(D_HID), _pair_spec(16),
              _full_spec((1, D_OUT_PAD)), _full_spec((D_HID, D_OUT_PAD))],
    out_specs=_rows_spec(D_OUT_PAD),
    out_shape=jax.ShapeDtypeStruct((N_PAD, D_OUT_PAD), jnp.float32),
)


def kernel(x, edge_index, W1, b1, W2, b2):
    xp = jnp.pad(x, ((0, N_PAD - N), (0, 0)))
    dst_d = edge_index[1].reshape(NW, NB_D, KB)
    src_a = edge_index[0].reshape(NUM_SUBCORES, NB_A, KB)
    dst_a = edge_index[1].reshape(NUM_SUBCORES, NB_A, KB)
    w2p = jnp.pad(W2, ((0, 0), (0, D_OUT_PAD - D_OUT)))
    b1r = b1.reshape(1, D_HID)
    b2r = jnp.pad(b2, (0, D_OUT_PAD - D_OUT)).reshape(1, D_OUT_PAD)

    degp = _deg(dst_d)
    m1p = _tc_a(xp, W1, degp)
    a1 = _agg(m1p, src_a, dst_a)
    hp = _tc_b(a1, m1p, degp, b1r)
    a2 = _agg(hp, src_a, dst_a)
    o48 = _tc_c(a2, hp, degp, b2r, w2p)
    return o48[:N, :D_OUT]


# speedup vs baseline: 2.2887x; 1.1435x over previous
import jax
import jax.numpy as jnp
from jax import lax
from jax.experimental import pallas as pl
from jax.experimental.pallas import tpu as pltpu
from jax.experimental.pallas import tpu_sc as plsc

N = 10000
N_PAD = 10240
E = 320000
D_IN = 128
D_HID = 128
D_OUT = 40
D_OUT_PAD = 48

NUM_CORES = 2
NUM_SUBCORES = 16
NW = NUM_CORES * NUM_SUBCORES

KB = 80

EPT_D = E // NW
NB_D = EPT_D // KB
RPT = N_PAD // NUM_SUBCORES
ZCH = 128

N_HALF = N_PAD // 2
ACC_ROWS = N_HALF + 16
EPT_A = E // NUM_SUBCORES
NB_A = EPT_A // KB
OPT = N_HALF // NUM_SUBCORES
ZCH_A = 16

_MESH = plsc.VectorSubcoreMesh(core_axis_name="c", subcore_axis_name="s")


def _zero_fill(ref, rows, cols):
    lanes = cols // 16
    zeros16 = jnp.zeros((16,), jnp.float32)

    def body(i, carry):
        r = i // lanes
        c = (i % lanes) * 16
        ref[r, pl.ds(c, 16)] = zeros16
        return carry

    lax.fori_loop(0, rows * lanes, body, 0)


def _deg_kernel(dst3, out, dst_v, ones_v, zbuf, acc, sem):
    cid = lax.axis_index("c")
    sid = lax.axis_index("s")
    wid = cid * NUM_SUBCORES + sid

    _zero_fill(zbuf, ZCH, 16)
    ones16 = jnp.ones((16,), jnp.float32)

    def fill_ones(i, carry):
        ones_v[i, :] = ones16
        return carry

    lax.fori_loop(0, KB, fill_ones, 0)

    row0 = sid * RPT

    def zchunk(j, carry):
        pltpu.sync_copy(zbuf, acc.at[pl.ds(row0 + j * ZCH, ZCH)])
        return carry

    lax.fori_loop(0, RPT // ZCH, zchunk, 0)
    plsc.subcore_barrier()

    pltpu.sync_copy(dst3.at[wid], dst_v)

    def batch(j, carry):
        pltpu.sync_copy(ones_v, acc.at[dst_v.at[j]], add=True)
        return carry

    lax.fori_loop(0, NB_D, batch, 0)
    plsc.subcore_barrier()

    def ochunk(j, carry):
        r = row0 + j * ZCH
        pltpu.sync_copy(acc.at[pl.ds(r, ZCH)], out.at[cid, pl.ds(r, ZCH)])
        return carry

    lax.fori_loop(0, RPT // ZCH, ochunk, 0)


def _agg_kernel(mp, src3, dst3, out, src_v, dst_v, dstp, rows_v, rows_w, zbuf, acc, sem, sem2):
    cid = lax.axis_index("c")
    sid = lax.axis_index("s")
    lo = cid * N_HALF
    trash = N_HALF + sid

    _zero_fill(zbuf, ZCH_A, D_HID)

    def zchunk(j, carry):
        pltpu.sync_copy(zbuf, acc.at[pl.ds((sid * 20 + j) * ZCH_A, ZCH_A)])
        return carry

    lax.fori_loop(0, N_HALF // ZCH_A // NUM_SUBCORES, zchunk, 0)

    @pl.when(sid == 0)
    def _():
        pltpu.sync_copy(zbuf.at[pl.ds(0, 16)], acc.at[pl.ds(N_HALF, 16)])

    plsc.subcore_barrier()

    pltpu.sync_copy(src3.at[sid], src_v)
    pltpu.sync_copy(dst3.at[sid], dst_v)

    def remap(j, dstp_b):
        for c in range(KB // 16):
            d = dst_v[j, pl.ds(c * 16, 16)]
            t = d - lo
            ok = (t >= 0) & (t < N_HALF)
            dstp_b[0, pl.ds(c * 16, 16)] = jnp.where(ok, t, trash)

    pltpu.async_copy(mp.at[src_v.at[0]], rows_v, sem).wait()

    def pair(m, carry):
        j0 = 2 * m
        dg1 = pltpu.async_copy(mp.at[src_v.at[j0 + 1]], rows_w, sem2)
        remap(j0, dstp)
        pltpu.sync_copy(rows_v, acc.at[dstp.at[0]], add=True)
        dg1.wait()

        @pl.when(j0 + 2 < NB_A)
        def _():
            pltpu.async_copy(mp.at[src_v.at[j0 + 2]], rows_v, sem).wait()

        remap(j0 + 1, dstp)
        pltpu.sync_copy(rows_w, acc.at[dstp.at[0]], add=True)
        return carry

    lax.fori_loop(0, NB_A // 2, pair, 0)
    plsc.subcore_barrier()

    r = sid * OPT
    pltpu.sync_copy(acc.at[pl.ds(r, OPT)], out.at[pl.ds(lo + r, OPT)])


_deg = pl.kernel(
    _deg_kernel,
    out_type=jax.ShapeDtypeStruct((NUM_CORES, N_PAD, 16), jnp.float32),
    mesh=_MESH,
    scratch_types=[
        pltpu.VMEM((NB_D, KB), jnp.int32),
        pltpu.VMEM((KB, 16), jnp.float32),
        pltpu.VMEM((ZCH, 16), jnp.float32),
        pltpu.VMEM_SHARED((N_PAD, 16), jnp.float32),
        pltpu.SemaphoreType.DMA,
    ],
)

_agg = pl.kernel(
    _agg_kernel,
    out_type=jax.ShapeDtypeStruct((N_PAD, D_HID), jnp.float32),
    mesh=_MESH,
    scratch_types=[
        pltpu.VMEM((NB_A, KB), jnp.int32),
        pltpu.VMEM((NB_A, KB), jnp.int32),
        pltpu.VMEM((8, KB), jnp.int32),
        pltpu.VMEM((KB, D_HID), jnp.float32),
        pltpu.VMEM((KB, D_HID), jnp.float32),
        pltpu.VMEM((ZCH_A, D_HID), jnp.float32),
        pltpu.VMEM_SHARED((ACC_ROWS, D_HID), jnp.float32),
        pltpu.SemaphoreType.DMA,
        pltpu.SemaphoreType.DMA,
    ],
)


BR = 1024


def _dis_from(degp_ref):
    deg = degp_ref[0, :, 0:1] + degp_ref[1, :, 0:1] + 1.0
    return lax.rsqrt(deg)


def _tc_a_body(x_ref, w_ref, degp_ref, m1p_ref):
    dis = _dis_from(degp_ref)
    m1p_ref[...] = dis * jnp.dot(x_ref[...], w_ref[...],
                                 preferred_element_type=jnp.float32)


def _tc_b_body(a1_ref, m1p_ref, degp_ref, b1_ref, hp_ref):
    dis = _dis_from(degp_ref)
    h = jnp.maximum(dis * (a1_ref[...] + m1p_ref[...]) + b1_ref[...], 0.0)
    hp_ref[...] = dis * h


def _tc_c_body(a2_ref, hp_ref, degp_ref, b2_ref, w2_ref, o_ref):
    dis = _dis_from(degp_ref)
    g = dis * (a2_ref[...] + hp_ref[...])
    o_ref[...] = jnp.dot(g, w2_ref[...], preferred_element_type=jnp.float32) \
        + b2_ref[...]


def _rows_spec(d):
    return pl.BlockSpec((BR, d), lambda i: (i, 0))


def _pair_spec(d):
    return pl.BlockSpec((2, BR, d), lambda i: (0, i, 0))


def _full_spec(shape):
    return pl.BlockSpec(shape, lambda i: tuple(0 for _ in shape))


_GRID = (N_PAD // BR,)

_tc_a = pl.pallas_call(
    _tc_a_body,
    grid=_GRID,
    in_specs=[_rows_spec(D_IN), _full_spec((D_IN, D_HID)), _pair_spec(16)],
    out_specs=_rows_spec(D_HID),
    out_shape=jax.ShapeDtypeStruct((N_PAD, D_HID), jnp.float32),
)

_tc_b = pl.pallas_call(
    _tc_b_body,
    grid=_GRID,
    in_specs=[_rows_spec(D_HID), _rows_spec(D_HID), _pair_spec(16),
              _full_spec((1, D_HID))],
    out_specs=_rows_spec(D_HID),
    out_shape=jax.ShapeDtypeStruct((N_PAD, D_HID), jnp.float32),
)

_tc_c = pl.pallas_call(
    _tc_c_body,
    grid=_GRID,
    in_specs=[_rows_spec(D_HID), _rows_spec(D_HID), _pair_spec(16),
              _full_spec((1, D_OUT_PAD)), _full_spec((D_HID, D_OUT_PAD))],
    out_specs=_rows_spec(D_OUT_PAD),
    out_shape=jax.ShapeDtypeStruct((N_PAD, D_OUT_PAD), jnp.float32),
)


def kernel(x, edge_index, W1, b1, W2, b2):
    xp = jnp.pad(x, ((0, N_PAD - N), (0, 0)))
    dst_d = edge_index[1].reshape(NW, NB_D, KB)
    src_a = edge_index[0].reshape(NUM_SUBCORES, NB_A, KB)
    dst_a = edge_index[1].reshape(NUM_SUBCORES, NB_A, KB)
    w2p = jnp.pad(W2, ((0, 0), (0, D_OUT_PAD - D_OUT)))
    b1r = b1.reshape(1, D_HID)
    b2r = jnp.pad(b2, (0, D_OUT_PAD - D_OUT)).reshape(1, D_OUT_PAD)

    degp = _deg(dst_d)
    m1p = _tc_a(xp, W1, degp)
    a1 = _agg(m1p, src_a, dst_a)
    hp = _tc_b(a1, m1p, degp, b1r)
    a2 = _agg(hp, src_a, dst_a)
    o48 = _tc_c(a2, hp, degp, b2r, w2p)
    return o48[:N, :D_OUT]


# speedup vs baseline: 2.6337x; 1.1508x over previous
import jax
import jax.numpy as jnp
from jax import lax
from jax.experimental import pallas as pl
from jax.experimental.pallas import tpu as pltpu
from jax.experimental.pallas import tpu_sc as plsc

N = 10000
N_PAD = 10240
E = 320000
D_IN = 128
D_HID = 128
D_OUT = 40
D_OUT_PAD = 48

NUM_CORES = 2
NUM_SUBCORES = 16
NW = NUM_CORES * NUM_SUBCORES

KB = 80

EPT_D = E // NW
NB_D = EPT_D // KB
RPT = N_PAD // NUM_SUBCORES
ZCH = 128

N_HALF = N_PAD // 2
ACC_ROWS = N_HALF + 16
EPT_A = E // NUM_SUBCORES
NB_A = EPT_A // KB
OPT = N_HALF // NUM_SUBCORES
ZCH_A = 16

_MESH = plsc.VectorSubcoreMesh(core_axis_name="c", subcore_axis_name="s")


def _zero_fill(ref, rows, cols):
    lanes = cols // 16
    zeros16 = jnp.zeros((16,), jnp.float32)

    def body(i, carry):
        r = i // lanes
        c = (i % lanes) * 16
        ref[r, pl.ds(c, 16)] = zeros16
        return carry

    lax.fori_loop(0, rows * lanes, body, 0)


def _deg_kernel(dst3, out, dst_v, ones_v, zbuf, acc, sem):
    cid = lax.axis_index("c")
    sid = lax.axis_index("s")
    wid = cid * NUM_SUBCORES + sid

    _zero_fill(zbuf, ZCH, 16)
    ones16 = jnp.ones((16,), jnp.float32)

    def fill_ones(i, carry):
        ones_v[i, :] = ones16
        return carry

    lax.fori_loop(0, KB, fill_ones, 0)

    row0 = sid * RPT

    def zchunk(j, carry):
        pltpu.sync_copy(zbuf, acc.at[pl.ds(row0 + j * ZCH, ZCH)])
        return carry

    lax.fori_loop(0, RPT // ZCH, zchunk, 0)
    plsc.subcore_barrier()

    pltpu.sync_copy(dst3.at[wid], dst_v)

    def batch(j, carry):
        pltpu.sync_copy(ones_v, acc.at[dst_v.at[j]], add=True)
        return carry

    lax.fori_loop(0, NB_D, batch, 0)
    plsc.subcore_barrier()

    def ochunk(j, carry):
        r = row0 + j * ZCH
        pltpu.sync_copy(acc.at[pl.ds(r, ZCH)], out.at[cid, pl.ds(r, ZCH)])
        return carry

    lax.fori_loop(0, RPT // ZCH, ochunk, 0)


def _agg_kernel(mp, src3, dst3, out, src_v, dst_v, dstp, rows_v, rows_w, zbuf, acc, sem, sem2):
    cid = lax.axis_index("c")
    sid = lax.axis_index("s")
    lo = cid * N_HALF
    trash = N_HALF + sid

    _zero_fill(zbuf, ZCH_A, D_HID)

    def zchunk(j, carry):
        pltpu.sync_copy(zbuf, acc.at[pl.ds((sid * 20 + j) * ZCH_A, ZCH_A)])
        return carry

    lax.fori_loop(0, N_HALF // ZCH_A // NUM_SUBCORES, zchunk, 0)

    @pl.when(sid == 0)
    def _():
        pltpu.sync_copy(zbuf.at[pl.ds(0, 16)], acc.at[pl.ds(N_HALF, 16)])

    plsc.subcore_barrier()

    pltpu.sync_copy(src3.at[sid], src_v)
    pltpu.sync_copy(dst3.at[sid], dst_v)

    def remap(j, dstp_b):
        for c in range(KB // 16):
            d = dst_v[j, pl.ds(c * 16, 16)]
            t = d - lo
            ok = (t >= 0) & (t < N_HALF)
            dstp_b[0, pl.ds(c * 16, 16)] = jnp.where(ok, t, trash)

    pltpu.async_copy(mp.at[src_v.at[0]], rows_v, sem).wait()

    def pair(m, carry):
        j0 = 2 * m
        dg1 = pltpu.async_copy(mp.at[src_v.at[j0 + 1]], rows_w, sem2)
        remap(j0, dstp)
        pltpu.sync_copy(rows_v, acc.at[dstp.at[0]], add=True)
        dg1.wait()

        @pl.when(j0 + 2 < NB_A)
        def _():
            dg2 = pltpu.async_copy(mp.at[src_v.at[j0 + 2]], rows_v, sem)
            remap(j0 + 1, dstp)
            pltpu.sync_copy(rows_w, acc.at[dstp.at[0]], add=True)
            dg2.wait()

        @pl.when(j0 + 2 >= NB_A)
        def _():
            remap(j0 + 1, dstp)
            pltpu.sync_copy(rows_w, acc.at[dstp.at[0]], add=True)

        return carry

    lax.fori_loop(0, NB_A // 2, pair, 0)
    plsc.subcore_barrier()

    r = sid * OPT
    pltpu.sync_copy(acc.at[pl.ds(r, OPT)], out.at[pl.ds(lo + r, OPT)])


_deg = pl.kernel(
    _deg_kernel,
    out_type=jax.ShapeDtypeStruct((NUM_CORES, N_PAD, 16), jnp.float32),
    mesh=_MESH,
    scratch_types=[
        pltpu.VMEM((NB_D, KB), jnp.int32),
        pltpu.VMEM((KB, 16), jnp.float32),
        pltpu.VMEM((ZCH, 16), jnp.float32),
        pltpu.VMEM_SHARED((N_PAD, 16), jnp.float32),
        pltpu.SemaphoreType.DMA,
    ],
)

_agg = pl.kernel(
    _agg_kernel,
    out_type=jax.ShapeDtypeStruct((N_PAD, D_HID), jnp.float32),
    mesh=_MESH,
    scratch_types=[
        pltpu.VMEM((NB_A, KB), jnp.int32),
        pltpu.VMEM((NB_A, KB), jnp.int32),
        pltpu.VMEM((8, KB), jnp.int32),
        pltpu.VMEM((KB, D_HID), jnp.float32),
        pltpu.VMEM((KB, D_HID), jnp.float32),
        pltpu.VMEM((ZCH_A, D_HID), jnp.float32),
        pltpu.VMEM_SHARED((ACC_ROWS, D_HID), jnp.float32),
        pltpu.SemaphoreType.DMA,
        pltpu.SemaphoreType.DMA,
    ],
)


BR = 1024


def _dis_from(degp_ref):
    deg = degp_ref[0, :, 0:1] + degp_ref[1, :, 0:1] + 1.0
    return lax.rsqrt(deg)


def _tc_a_body(x_ref, w_ref, degp_ref, m1p_ref):
    dis = _dis_from(degp_ref)
    m1p_ref[...] = dis * jnp.dot(x_ref[...], w_ref[...],
                                 preferred_element_type=jnp.float32)


def _tc_b_body(a1_ref, m1p_ref, degp_ref, b1_ref, hp_ref):
    dis = _dis_from(degp_ref)
    h = jnp.maximum(dis * (a1_ref[...] + m1p_ref[...]) + b1_ref[...], 0.0)
    hp_ref[...] = dis * h


def _tc_c_body(a2_ref, hp_ref, degp_ref, b2_ref, w2_ref, o_ref):
    dis = _dis_from(degp_ref)
    g = dis * (a2_ref[...] + hp_ref[...])
    o_ref[...] = jnp.dot(g, w2_ref[...], preferred_element_type=jnp.float32) \
        + b2_ref[...]


def _rows_spec(d):
    return pl.BlockSpec((BR, d), lambda i: (i, 0))


def _pair_spec(d):
    return pl.BlockSpec((2, BR, d), lambda i: (0, i, 0))


def _full_spec(shape):
    return pl.BlockSpec(shape, lambda i: tuple(0 for _ in shape))


_GRID = (N_PAD // BR,)

_tc_a = pl.pallas_call(
    _tc_a_body,
    grid=_GRID,
    in_specs=[_rows_spec(D_IN), _full_spec((D_IN, D_HID)), _pair_spec(16)],
    out_specs=_rows_spec(D_HID),
    out_shape=jax.ShapeDtypeStruct((N_PAD, D_HID), jnp.float32),
)

_tc_b = pl.pallas_call(
    _tc_b_body,
    grid=_GRID,
    in_specs=[_rows_spec(D_HID), _rows_spec(D_HID), _pair_spec(16),
              _full_spec((1, D_HID))],
    out_specs=_rows_spec(D_HID),
    out_shape=jax.ShapeDtypeStruct((N_PAD, D_HID), jnp.float32),
)

_tc_c = pl.pallas_call(
    _tc_c_body,
    grid=_GRID,
    in_specs=[_rows_spec(D_HID), _rows_spec(D_HID), _pair_spec(16),
              _full_spec((1, D_OUT_PAD)), _full_spec((D_HID, D_OUT_PAD))],
    out_specs=_rows_spec(D_OUT_PAD),
    out_shape=jax.ShapeDtypeStruct((N_PAD, D_OUT_PAD), jnp.float32),
)


def kernel(x, edge_index, W1, b1, W2, b2):
    xp = jnp.pad(x, ((0, N_PAD - N), (0, 0)))
    dst_d = edge_index[1].reshape(NW, NB_D, KB)
    src_a = edge_index[0].reshape(NUM_SUBCORES, NB_A, KB)
    dst_a = edge_index[1].reshape(NUM_SUBCORES, NB_A, KB)
    w2p = jnp.pad(W2, ((0, 0), (0, D_OUT_PAD - D_OUT)))
    b1r = b1.reshape(1, D_HID)
    b2r = jnp.pad(b2, (0, D_OUT_PAD - D_OUT)).reshape(1, D_OUT_PAD)

    degp = _deg(dst_d)
    m1p = _tc_a(xp, W1, degp)
    a1 = _agg(m1p, src_a, dst_a)
    hp = _tc_b(a1, m1p, degp, b1r)
    a2 = _agg(hp, src_a, dst_a)
    o48 = _tc_c(a2, hp, degp, b2r, w2p)
    return o48[:N, :D_OUT]
